# baseline (speedup 1.0000x reference)
import functools
import math

import jax
import jax.numpy as jnp
from jax.experimental import pallas as pl

N = 16384
B = 16
MAX_NODE = 2048
E = 524288
D = 128
H = 128
OUT = 10
L = 5


def _mlp_body(x_ref, w1_ref, b1_ref, w2_ref, b2_ref, o_ref):
    x = x_ref[...]
    z = jnp.maximum(jnp.dot(x, w1_ref[...], preferred_element_type=jnp.float32) + b1_ref[...], 0.0)
    o_ref[...] = jnp.maximum(jnp.dot(z, w2_ref[...], preferred_element_type=jnp.float32) + b2_ref[...], 0.0)


def _mlp(x, W1, b1, W2, b2):
    blk = 512
    grid = (N // blk,)
    return pl.pallas_call(
        _mlp_body,
        grid=grid,
        in_specs=[
            pl.BlockSpec((blk, D), lambda i: (i, 0)),
            pl.BlockSpec((D, H), lambda i: (0, 0)),
            pl.BlockSpec((1, H), lambda i: (0, 0)),
            pl.BlockSpec((H, H), lambda i: (0, 0)),
            pl.BlockSpec((1, H), lambda i: (0, 0)),
        ],
        out_specs=pl.BlockSpec((blk, H), lambda i: (i, 0)),
        out_shape=jax.ShapeDtypeStruct((N, H), jnp.float32),
    )(x, W1, b1.reshape(1, H), W2, b2.reshape(1, H))


def _sine_embed(pos, num_pos_feats=128, temperature=10000.0, eps=1e-6):
    scale = 2.0 * math.pi
    emb = pos / (jnp.max(pos, axis=1, keepdims=True) + eps) * scale
    dim_t = temperature ** (2.0 * (jnp.arange(num_pos_feats) // 2) / num_pos_feats)
    p = emb[..., None] / dim_t
    p = jnp.stack([jnp.sin(p[..., 0::2]), jnp.cos(p[..., 1::2])], axis=-1)
    return p.reshape(pos.shape[0], pos.shape[1], num_pos_feats)


def kernel(h, edge_index, centroid, node_offsets, W1s, b1s, W2s, b2s, predW, predb):
    src = edge_index[0]
    dst = edge_index[1]
    deg = jnp.maximum(jnp.zeros((N,), jnp.float32).at[dst].add(1.0), 1.0)
    inv_deg = 1.0 / deg
    reps = [h]
    cur = h
    for l in range(L - 1):
        agg = jnp.zeros((N, H), jnp.float32).at[dst].add(cur[src]) * inv_deg[:, None]
        cur = _mlp(cur + agg, W1s[l], b1s[l], W2s[l], b2s[l])
        reps.append(cur)
    ids = jnp.searchsorted(node_offsets, jnp.arange(N), side='right') - 1
    counts = node_offsets[1:] - node_offsets[:-1]
    cs = jnp.maximum(counts.astype(jnp.float32), 1.0)
    score = jnp.zeros((B, OUT), jnp.float32)
    for l in range(L):
        pooled = jnp.zeros((B, H), jnp.float32).at[ids].add(reps[l]) / cs[:, None]
        score = score + pooled @ predW[l] + predb[l]
    semantic = jnp.zeros((B, H), jnp.float32).at[ids].add(reps[-1]) / cs[:, None]
    pos_in_graph = jnp.arange(N) - node_offsets[ids]
    valid = pos_in_graph < MAX_NODE
    safe = jnp.where(valid, ids * MAX_NODE + pos_in_graph, B * MAX_NODE)
    features = jnp.zeros((B * MAX_NODE + 1, D), jnp.float32).at[safe].add(h)[:-1].reshape(B, MAX_NODE, D)
    pos_x = jnp.zeros((B * MAX_NODE + 1,), jnp.float32).at[safe].add(centroid[:, 0])[:-1].reshape(B, MAX_NODE)
    pos_y = jnp.zeros((B * MAX_NODE + 1,), jnp.float32).at[safe].add(centroid[:, 1])[:-1].reshape(B, MAX_NODE)
    mask = jnp.where((counts[:, None] <= MAX_NODE) & (jnp.arange(MAX_NODE)[None, :] >= counts[:, None]), 1.0, 0.0)
    pos_emd = jnp.concatenate([_sine_embed(pos_y), _sine_embed(pos_x)], axis=-1)
    return (features, mask, pos_emd, score, semantic)


# speedup vs baseline: 5.3940x; 5.3940x over previous
import functools
import math

import jax
import jax.numpy as jnp
from jax import lax
from jax.experimental import pallas as pl
from jax.experimental.pallas import tpu as pltpu
from jax.experimental.pallas import tpu_sc as plsc

N = 16384
B = 16
MAX_NODE = 2048
E = 524288
D = 128
H = 128
OUT = 10
L = 5

HALF = D // 2
ECHUNK = 128
NS = 16
CHUNKS_PER_TILE = (E // ECHUNK) // NS
ROWS_PER_TILE = N // NS


NPH = 4
CPP = CHUNKS_PER_TILE // NPH


def _agg_body(cur_lo, cur_hi, src2, dst2, zrows,
              agg_lo, agg_hi, src_all, dst_all, rows, acc):
    c = lax.axis_index("c")
    s = lax.axis_index("s")
    tbase = s * CHUNKS_PER_TILE

    pltpu.sync_copy(zrows, rows)
    for k in range(ROWS_PER_TILE // ECHUNK):
        pltpu.sync_copy(rows, acc.at[pl.ds(s * ROWS_PER_TILE + k * ECHUNK, ECHUNK), :])
    plsc.subcore_barrier()

    def run_core(cur_ref):
        for phase in range(NPH):
            pbase = tbase + phase * CPP
            pltpu.sync_copy(src2.at[pl.ds(pbase, CPP), :], src_all)
            pltpu.sync_copy(dst2.at[pl.ds(pbase, CPP), :], dst_all)

            def body(j, carry):
                pltpu.sync_copy(cur_ref.at[src_all.at[j]], rows)
                pltpu.sync_copy(rows, acc.at[dst_all.at[j]], add=True)
                return carry
            lax.fori_loop(0, CPP, body, 0)

    @pl.when(c == 0)
    def _():
        run_core(cur_lo)

    @pl.when(c == 1)
    def _():
        run_core(cur_hi)

    plsc.subcore_barrier()

    rslice = pl.ds(s * ROWS_PER_TILE, ROWS_PER_TILE)

    @pl.when(c == 0)
    def _():
        pltpu.sync_copy(acc.at[rslice, :], agg_lo.at[rslice, :])

    @pl.when(c == 1)
    def _():
        pltpu.sync_copy(acc.at[rslice, :], agg_hi.at[rslice, :])


def _make_agg():
    mesh = plsc.VectorSubcoreMesh(core_axis_name="c", subcore_axis_name="s")
    return pl.kernel(
        _agg_body,
        mesh=mesh,
        out_type=(jax.ShapeDtypeStruct((N, HALF), jnp.float32),
                  jax.ShapeDtypeStruct((N, HALF), jnp.float32)),
        compiler_params=pltpu.CompilerParams(use_tc_tiling_on_sc=False),
        scratch_types=[
            pltpu.VMEM((CPP, ECHUNK), jnp.int32),
            pltpu.VMEM((CPP, ECHUNK), jnp.int32),
            pltpu.VMEM((ECHUNK, HALF), jnp.float32),
            pltpu.VMEM_SHARED((N, HALF), jnp.float32),
        ],
    )


_agg = _make_agg()

DEG_CHUNKS_PER_TILE = (E // ECHUNK) // (2 * NS)


def _deg_body(dst2, cdeg, deg0, deg1, dst_all, deg16, deg_acc):
    c = lax.axis_index("c")
    s = lax.axis_index("s")
    wid = c * NS + s

    tbase = wid * DEG_CHUNKS_PER_TILE
    pltpu.sync_copy(dst2.at[pl.ds(tbase, DEG_CHUNKS_PER_TILE), :], dst_all)

    pltpu.sync_copy(cdeg.at[0], deg16)
    for k in range(ROWS_PER_TILE // ECHUNK):
        pltpu.sync_copy(deg16, deg_acc.at[pl.ds(s * ROWS_PER_TILE + k * ECHUNK, ECHUNK), :])
    pltpu.sync_copy(cdeg.at[1], deg16)
    plsc.subcore_barrier()

    def body(j, carry):
        pltpu.sync_copy(deg16, deg_acc.at[dst_all.at[j]], add=True)
        return carry
    lax.fori_loop(0, DEG_CHUNKS_PER_TILE, body, 0)

    plsc.subcore_barrier()
    rslice = pl.ds(s * ROWS_PER_TILE, ROWS_PER_TILE)

    @pl.when(c == 0)
    def _():
        pltpu.sync_copy(deg_acc.at[rslice, :], deg0.at[rslice, :])

    @pl.when(c == 1)
    def _():
        pltpu.sync_copy(deg_acc.at[rslice, :], deg1.at[rslice, :])


def _make_deg():
    mesh = plsc.VectorSubcoreMesh(core_axis_name="c", subcore_axis_name="s")
    return pl.kernel(
        _deg_body,
        mesh=mesh,
        out_type=(jax.ShapeDtypeStruct((N, 16), jnp.float32),
                  jax.ShapeDtypeStruct((N, 16), jnp.float32)),
        compiler_params=pltpu.CompilerParams(use_tc_tiling_on_sc=False),
        scratch_types=[
            pltpu.VMEM((DEG_CHUNKS_PER_TILE, ECHUNK), jnp.int32),
            pltpu.VMEM((ECHUNK, 16), jnp.float32),
            pltpu.VMEM_SHARED((N, 16), jnp.float32),
        ],
    )


_deg = _make_deg()


def _mlp_body(clo_ref, chi_ref, alo_ref, ahi_ref, deg_ref,
              w1_ref, b1_ref, w2_ref, b2_ref, olo_ref, ohi_ref):
    x = jnp.concatenate([clo_ref[...], chi_ref[...]], axis=1)
    a = jnp.concatenate([alo_ref[...], ahi_ref[...]], axis=1)
    dinv = 1.0 / jnp.maximum(deg_ref[...], 1.0)
    x = x + a * dinv
    z = jnp.maximum(jnp.dot(x, w1_ref[...], preferred_element_type=jnp.float32) + b1_ref[...], 0.0)
    z = jnp.maximum(jnp.dot(z, w2_ref[...], preferred_element_type=jnp.float32) + b2_ref[...], 0.0)
    olo_ref[...] = z[:, :HALF]
    ohi_ref[...] = z[:, HALF:]


def _mlp(clo, chi, alo, ahi, deg_col, W1, b1, W2, b2):
    blk = 512
    return pl.pallas_call(
        _mlp_body,
        grid=(N // blk,),
        in_specs=[
            pl.BlockSpec((blk, HALF), lambda i: (i, 0)),
            pl.BlockSpec((blk, HALF), lambda i: (i, 0)),
            pl.BlockSpec((blk, HALF), lambda i: (i, 0)),
            pl.BlockSpec((blk, HALF), lambda i: (i, 0)),
            pl.BlockSpec((blk, 1), lambda i: (i, 0)),
            pl.BlockSpec((D, H), lambda i: (0, 0)),
            pl.BlockSpec((1, H), lambda i: (0, 0)),
            pl.BlockSpec((H, H), lambda i: (0, 0)),
            pl.BlockSpec((1, H), lambda i: (0, 0)),
        ],
        out_specs=(pl.BlockSpec((blk, HALF), lambda i: (i, 0)),
                   pl.BlockSpec((blk, HALF), lambda i: (i, 0))),
        out_shape=(jax.ShapeDtypeStruct((N, HALF), jnp.float32),
                   jax.ShapeDtypeStruct((N, HALF), jnp.float32)),
    )(clo, chi, alo, ahi, deg_col, W1, b1.reshape(1, H), W2, b2.reshape(1, H))


def _sine_embed(pos, num_pos_feats=128, temperature=10000.0, eps=1e-6):
    scale = 2.0 * math.pi
    emb = pos / (jnp.max(pos, axis=1, keepdims=True) + eps) * scale
    dim_t = temperature ** (2.0 * (jnp.arange(num_pos_feats) // 2) / num_pos_feats)
    p = emb[..., None] / dim_t
    p = jnp.stack([jnp.sin(p[..., 0::2]), jnp.cos(p[..., 1::2])], axis=-1)
    return p.reshape(pos.shape[0], pos.shape[1], num_pos_feats)


def kernel(h, edge_index, centroid, node_offsets, W1s, b1s, W2s, b2s, predW, predb):
    src2 = edge_index[0].reshape(E // ECHUNK, ECHUNK)
    dst2 = edge_index[1].reshape(E // ECHUNK, ECHUNK)
    h_lo = h[:, :HALF]
    h_hi = h[:, HALF:]
    zrows = jnp.zeros((ECHUNK, HALF), jnp.float32)
    cdeg = jnp.stack([jnp.zeros((ECHUNK, 16), jnp.float32),
                      jnp.ones((ECHUNK, 16), jnp.float32)])

    d0, d1 = _deg(dst2, cdeg)
    deg_col = d0[:, :1] + d1[:, :1]

    reps = [h]
    cur_lo, cur_hi = h_lo, h_hi
    agg_lo = agg_hi = None
    for l in range(L - 1):
        agg_lo, agg_hi = _agg(cur_lo, cur_hi, src2, dst2, zrows)
        cur_lo, cur_hi = _mlp(cur_lo, cur_hi, agg_lo, agg_hi, deg_col,
                              W1s[l], b1s[l], W2s[l], b2s[l])
        reps.append(jnp.concatenate([cur_lo, cur_hi], axis=1))

    ids = jnp.searchsorted(node_offsets, jnp.arange(N), side='right') - 1
    counts = node_offsets[1:] - node_offsets[:-1]
    cs = jnp.maximum(counts.astype(jnp.float32), 1.0)
    score = jnp.zeros((B, OUT), jnp.float32)
    for l in range(L):
        pooled = jnp.zeros((B, H), jnp.float32).at[ids].add(reps[l]) / cs[:, None]
        score = score + pooled @ predW[l] + predb[l]
    semantic = jnp.zeros((B, H), jnp.float32).at[ids].add(reps[-1]) / cs[:, None]
    pos_in_graph = jnp.arange(N) - node_offsets[ids]
    valid = pos_in_graph < MAX_NODE
    safe = jnp.where(valid, ids * MAX_NODE + pos_in_graph, B * MAX_NODE)
    features = jnp.zeros((B * MAX_NODE + 1, D), jnp.float32).at[safe].add(h)[:-1].reshape(B, MAX_NODE, D)
    pos_x = jnp.zeros((B * MAX_NODE + 1,), jnp.float32).at[safe].add(centroid[:, 0])[:-1].reshape(B, MAX_NODE)
    pos_y = jnp.zeros((B * MAX_NODE + 1,), jnp.float32).at[safe].add(centroid[:, 1])[:-1].reshape(B, MAX_NODE)
    mask = jnp.where((counts[:, None] <= MAX_NODE) & (jnp.arange(MAX_NODE)[None, :] >= counts[:, None]), 1.0, 0.0)
    pos_emd = jnp.concatenate([_sine_embed(pos_y), _sine_embed(pos_x)], axis=-1)
    return (features, mask, pos_emd, score, semantic)
